# baseline (speedup 1.0000x reference)
import functools

import jax
import jax.numpy as jnp
import numpy as np
from jax import lax
from jax.experimental import pallas as pl
from jax.experimental.pallas import tpu as pltpu
from jax.experimental.pallas import tpu_sc as plsc

_SR = 2
_S = 2 * _SR + 1
_N_IN = _S * _S


def _spiral_prio():
    coords = [(0, 0)]
    j = i = 0
    step = 1
    dirs = [(0, 1), (1, 0), (0, -1), (-1, 0)]
    d = 0
    while len(coords) < _S * _S:
        for _ in range(2):
            dj, di = dirs[d]
            for _ in range(step):
                j += dj
                i += di
                if abs(j) <= _SR and abs(i) <= _SR and len(coords) < _S * _S:
                    coords.append((j, i))
            d = (d + 1) % 4
        step += 1
    order = [(jj + _SR) * _S + (ii + _SR) for jj, ii in coords]
    prio = [0] * _N_IN
    for r, n in enumerate(order):
        prio[n] = r
    return prio


_PRIO = _spiral_prio()

_L = 16
_NW = 32
_CP = 128


def _body(pcode_hbm, w1_hbm, w2_hbm, vy_o, vx_o, msk_o, mcv_o, tmp_o,
          pcode_v, w1_v, w2_v, vy_v, vx_v, msk_v, mcv_v, tmp_v,
          *, n_cand, ppw):
    rec = n_cand * _L
    wid = lax.axis_index("s") * 2 + lax.axis_index("c")
    iota = lax.broadcasted_iota(jnp.int32, (_L,), 0)
    pltpu.sync_copy(pcode_hbm, pcode_v)

    def chunk(c, _):
        base_px = wid * ppw + c * _CP
        pltpu.sync_copy(w1_hbm.at[pl.ds(base_px * rec, _CP * rec)], w1_v)
        pltpu.sync_copy(w2_hbm.at[pl.ds(base_px * rec, _CP * rec)], w2_v)

        def pgroup(g, _):
            q0 = g * _L
            pbase = (iota + q0) * rec

            def cost_of(n):
                pb = pbase + n * _L
                acc = jnp.zeros((_L,), jnp.float32)
                for k in range(_L):
                    idx = pb + ((iota + k) & (_L - 1))
                    a = plsc.load_gather(w1_v, [idx])
                    b = plsc.load_gather(w2_v, [idx])
                    acc = acc + jnp.abs(a - b)
                return acc.astype(jnp.int32)

            def nbody(n, best):
                code = plsc.load_gather(
                    pcode_v, [jnp.broadcast_to(n, (_L,))])
                key = (cost_of(n) << 10) | code
                return jnp.minimum(best, key)

            best = lax.fori_loop(
                0, _N_IN, nbody, jnp.full((_L,), 1 << 30, jnp.int32))
            n_bm = best & 31
            mcb = best >> 10
            if n_cand > _N_IN:
                c25 = cost_of(_N_IN)
                hit = c25 < mcb
                mskv = hit.astype(jnp.int32)
                mcv = jnp.minimum(c25, mcb)
                idx_tm = jnp.where(hit, _N_IN, n_bm)
            else:
                mskv = jnp.zeros((_L,), jnp.int32)
                mcv = mcb
                idx_tm = n_bm
            row = (n_bm * 13) >> 6
            col = n_bm - row * 5
            sl = pl.ds(q0, _L)
            vy_v[sl] = (2 - row).astype(jnp.float32)
            vx_v[sl] = (2 - col).astype(jnp.float32)
            msk_v[sl] = mskv
            mcv_v[sl] = mcv
            tb = pbase + idx_tm * _L
            ob = (iota + q0) * _L
            for k in range(_L):
                sk = (iota + k) & (_L - 1)
                val = plsc.load_gather(w1_v, [tb + sk])
                plsc.store_scatter(tmp_v, [ob + sk], val)
            return ()

        lax.fori_loop(0, _CP // _L, pgroup, ())
        pltpu.sync_copy(vy_v, vy_o.at[pl.ds(base_px, _CP)])
        pltpu.sync_copy(vx_v, vx_o.at[pl.ds(base_px, _CP)])
        pltpu.sync_copy(msk_v, msk_o.at[pl.ds(base_px, _CP)])
        pltpu.sync_copy(mcv_v, mcv_o.at[pl.ds(base_px, _CP)])
        pltpu.sync_copy(tmp_v, tmp_o.at[pl.ds(base_px * _L, _CP * _L)])
        return ()

    lax.fori_loop(0, ppw // _CP, chunk, ())


@jax.jit
def kernel(w1, w2):
    B, H, W, N, K2 = w1.shape
    P = B * H * W
    ppw = P // _NW
    rec = N * K2
    mesh = plsc.VectorSubcoreMesh(
        core_axis_name="c", subcore_axis_name="s", num_cores=2, num_subcores=16
    )
    f32 = jnp.float32
    i32 = jnp.int32
    out_type = (
        jax.ShapeDtypeStruct((P,), f32),
        jax.ShapeDtypeStruct((P,), f32),
        jax.ShapeDtypeStruct((P,), i32),
        jax.ShapeDtypeStruct((P,), i32),
        jax.ShapeDtypeStruct((P * K2,), f32),
    )
    scratch = (
        pltpu.VMEM((_L * 2,), i32),
        pltpu.VMEM((_CP * rec,), f32),
        pltpu.VMEM((_CP * rec,), f32),
        pltpu.VMEM((_CP,), f32),
        pltpu.VMEM((_CP,), f32),
        pltpu.VMEM((_CP,), i32),
        pltpu.VMEM((_CP,), i32),
        pltpu.VMEM((_CP * K2,), f32),
    )
    run = pl.kernel(
        functools.partial(_body, n_cand=N, ppw=ppw),
        out_type=out_type,
        mesh=mesh,
        scratch_types=scratch,
        compiler_params=pltpu.CompilerParams(needs_layout_passes=False),
    )
    pcode = np.full((_L * 2,), 1 << 30, np.int32)
    for n in range(_N_IN):
        pcode[n] = (_PRIO[n] << 5) | n
    vy, vx, msk, mcv, tmp = run(
        jnp.asarray(pcode), w1.reshape(-1), w2.reshape(-1))
    vector = jnp.stack([vy, vx], axis=-1).reshape(B, H, W, 2)
    vector = vector.astype(jnp.float16)
    min_templates = tmp.reshape(B, H, W, 1, K2)
    input_mv_mask = (msk > 0).reshape(B, H, W, 1)
    min_cost_volume = mcv.reshape(B, H, W, 1)
    return (vector, min_templates, input_mv_mask, min_cost_volume)


# speedup vs baseline: 8.2885x; 8.2885x over previous
import functools

import jax
import jax.numpy as jnp
import numpy as np
from jax import lax
from jax.experimental import pallas as pl
from jax.experimental.pallas import tpu as pltpu
from jax.experimental.pallas import tpu_sc as plsc

_SR = 2
_S = 2 * _SR + 1
_N_IN = _S * _S


def _spiral_prio():
    coords = [(0, 0)]
    j = i = 0
    step = 1
    dirs = [(0, 1), (1, 0), (0, -1), (-1, 0)]
    d = 0
    while len(coords) < _S * _S:
        for _ in range(2):
            dj, di = dirs[d]
            for _ in range(step):
                j += dj
                i += di
                if abs(j) <= _SR and abs(i) <= _SR and len(coords) < _S * _S:
                    coords.append((j, i))
            d = (d + 1) % 4
        step += 1
    order = [(jj + _SR) * _S + (ii + _SR) for jj, ii in coords]
    prio = [0] * _N_IN
    for r, n in enumerate(order):
        prio[n] = r
    return prio


_PRIO = _spiral_prio()

_L = 16
_NW = 32


def _body(pcode_hbm, w1_hbm, w2_hbm, vv_o, msk_o, mcv_o, tmp_o,
          pcode_v, w1_v, w2_v, vv_v, msk_v, mcv_v, tmp_v,
          *, n_cand, k2, w, rows_per_tec):
    slab = n_cand * k2 * w
    wid = lax.axis_index("s") * 2 + lax.axis_index("c")
    iota = lax.broadcasted_iota(jnp.int32, (_L,), 0)
    pltpu.sync_copy(pcode_hbm, pcode_v)

    def row(r, _):
        bh = wid * rows_per_tec + r
        pltpu.sync_copy(w1_hbm.at[pl.ds(bh * slab, slab)], w1_v)
        pltpu.sync_copy(w2_hbm.at[pl.ds(bh * slab, slab)], w2_v)

        def pgroup(g, _):
            w0 = g * _L

            def nbody(n, best):
                code = plsc.load_gather(
                    pcode_v, [jnp.broadcast_to(n, (_L,))])
                acc = code
                base = n * (k2 * w) + w0
                for k in range(k2):
                    sl = pl.ds(base + k * w, _L)
                    acc = acc + jnp.abs(w1_v[sl] - w2_v[sl])
                return jnp.minimum(best, acc)

            best = lax.fori_loop(
                0, _N_IN, nbody, jnp.full((_L,), 3.0e7, jnp.float32))
            best_i = (best * 1024.0).astype(jnp.int32)
            n_bm = best_i & 31
            mcb = best_i >> 10
            if n_cand > _N_IN:
                acc = jnp.zeros((_L,), jnp.float32)
                base = _N_IN * (k2 * w) + w0
                for k in range(k2):
                    sl = pl.ds(base + k * w, _L)
                    acc = acc + jnp.abs(w1_v[sl] - w2_v[sl])
                c25 = acc.astype(jnp.int32)
                hit = c25 < mcb
                mskv = hit.astype(jnp.int32)
                mcv = jnp.minimum(c25, mcb)
                idx_tm = jnp.where(hit, _N_IN, n_bm)
            else:
                mskv = jnp.zeros((_L,), jnp.int32)
                mcv = mcb
                idx_tm = n_bm
            row_i = (n_bm * 13) >> 6
            col_i = n_bm - row_i * 5
            sl = pl.ds(w0, _L)
            vv_v[sl] = (2 - row_i).astype(jnp.float32)
            vv_v[pl.ds(w + w0, _L)] = (2 - col_i).astype(jnp.float32)
            msk_v[sl] = mskv
            mcv_v[sl] = mcv
            tb = idx_tm * (k2 * w) + (w0 + iota)
            for k in range(k2):
                val = plsc.load_gather(w1_v, [tb + k * w])
                tmp_v[pl.ds(k * w + w0, _L)] = val
            return ()

        lax.fori_loop(0, w // _L, pgroup, ())
        pltpu.sync_copy(vv_v, vv_o.at[pl.ds(bh * 2 * w, 2 * w)])
        pltpu.sync_copy(msk_v, msk_o.at[pl.ds(bh * w, w)])
        pltpu.sync_copy(mcv_v, mcv_o.at[pl.ds(bh * w, w)])
        pltpu.sync_copy(tmp_v, tmp_o.at[pl.ds(bh * k2 * w, k2 * w)])
        return ()

    lax.fori_loop(0, rows_per_tec, row, ())


@jax.jit
def kernel(w1, w2):
    B, H, W, N, K2 = w1.shape
    BH = B * H
    rows_per_tec = BH // _NW
    mesh = plsc.VectorSubcoreMesh(
        core_axis_name="c", subcore_axis_name="s", num_cores=2, num_subcores=16
    )
    f32 = jnp.float32
    i32 = jnp.int32
    out_type = (
        jax.ShapeDtypeStruct((BH * 2 * W,), f32),
        jax.ShapeDtypeStruct((BH * W,), i32),
        jax.ShapeDtypeStruct((BH * W,), i32),
        jax.ShapeDtypeStruct((BH * K2 * W,), f32),
    )
    slab = N * K2 * W
    scratch = (
        pltpu.VMEM((_L * 2,), f32),
        pltpu.VMEM((slab,), f32),
        pltpu.VMEM((slab,), f32),
        pltpu.VMEM((2 * W,), f32),
        pltpu.VMEM((W,), i32),
        pltpu.VMEM((W,), i32),
        pltpu.VMEM((K2 * W,), f32),
    )
    run = pl.kernel(
        functools.partial(_body, n_cand=N, k2=K2, w=W,
                          rows_per_tec=rows_per_tec),
        out_type=out_type,
        mesh=mesh,
        scratch_types=scratch,
        compiler_params=pltpu.CompilerParams(needs_layout_passes=False),
    )
    pcode = np.full((_L * 2,), (1 << 22), np.float32)
    for n in range(_N_IN):
        pcode[n] = ((_PRIO[n] << 5) | n) / 1024.0
    w1t = jnp.transpose(w1, (0, 1, 3, 4, 2)).reshape(-1)
    w2t = jnp.transpose(w2, (0, 1, 3, 4, 2)).reshape(-1)
    vv, msk, mcv, tmp = run(jnp.asarray(pcode), w1t, w2t)
    vector = vv.reshape(B, H, 2, W).transpose(0, 1, 3, 2).astype(jnp.float16)
    min_templates = tmp.reshape(B, H, 1, K2, W).transpose(0, 1, 4, 2, 3)
    input_mv_mask = (msk > 0).reshape(B, H, W, 1)
    min_cost_volume = mcv.reshape(B, H, W, 1)
    return (vector, min_templates, input_mv_mask, min_cost_volume)


# speedup vs baseline: 10.5392x; 1.2716x over previous
import functools

import jax
import jax.numpy as jnp
import numpy as np
from jax import lax
from jax.experimental import pallas as pl
from jax.experimental.pallas import tpu as pltpu
from jax.experimental.pallas import tpu_sc as plsc

_SR = 2
_S = 2 * _SR + 1
_N_IN = _S * _S


def _spiral_prio():
    coords = [(0, 0)]
    j = i = 0
    step = 1
    dirs = [(0, 1), (1, 0), (0, -1), (-1, 0)]
    d = 0
    while len(coords) < _S * _S:
        for _ in range(2):
            dj, di = dirs[d]
            for _ in range(step):
                j += dj
                i += di
                if abs(j) <= _SR and abs(i) <= _SR and len(coords) < _S * _S:
                    coords.append((j, i))
            d = (d + 1) % 4
        step += 1
    order = [(jj + _SR) * _S + (ii + _SR) for jj, ii in coords]
    prio = [0] * _N_IN
    for r, n in enumerate(order):
        prio[n] = r
    return prio


_PRIO = _spiral_prio()

_L = 16
_NW = 32


_NA = 13


def _body(pcode_hbm, w1_hbm, w2_hbm, vv_o, msk_o, mcv_o, tmp_o,
          pcode_v, w1a, w2a, w1b, w2b, best_v, vv_v, msk_v, mcv_v, tmp_v,
          sem_a, sem_b,
          *, n_cand, k2, w, rows_per_tec):
    nr = _NA * k2
    wid = lax.axis_index("s") * 2 + lax.axis_index("c")
    iota = lax.broadcasted_iota(jnp.int32, (_L,), 0)
    pltpu.sync_copy(pcode_hbm, pcode_v)

    def start_in(c, b1, b2, sem):
        bh = wid * rows_per_tec + (c >> 1)
        r0 = (c & 1) * nr
        pltpu.make_async_copy(
            w1_hbm.at[bh, pl.ds(r0, nr), :], b1, sem).start()
        pltpu.make_async_copy(
            w2_hbm.at[bh, pl.ds(r0, nr), :], b2, sem).start()

    def wait_in(b1, b2, sem):
        pltpu.make_async_copy(
            w1_hbm.at[0, pl.ds(0, nr), :], b1, sem).wait()
        pltpu.make_async_copy(
            w2_hbm.at[0, pl.ds(0, nr), :], b2, sem).wait()

    def cost_min(b1, b2, wl, n_lo, n_hi, init):
        def nbody(n, best):
            code = plsc.load_gather(pcode_v, [jnp.broadcast_to(n, (_L,))])
            acc = code
            r = (n - n_lo) * k2
            for k in range(k2):
                acc = acc + jnp.abs(b1[r + k, pl.ds(wl, _L)]
                                    - b2[r + k, pl.ds(wl, _L)])
            return jnp.minimum(best, acc)

        return lax.fori_loop(n_lo, n_hi, nbody, init)

    def compute_a(b1, b2):
        for g in range(w // _L):
            wl = g * _L
            best = cost_min(b1, b2, wl, 0, _NA,
                            jnp.full((_L,), 3.0e7, jnp.float32))
            best_v[pl.ds(wl, _L)] = best
            na = (best * 1024.0).astype(jnp.int32) & 31
            rows = na * k2
            cols = wl + iota
            for k in range(k2):
                val = plsc.load_gather(b1, [rows + k, cols])
                tmp_v[pl.ds(k * w + wl, _L)] = val

    def compute_b(b1, b2):
        for g in range(w // _L):
            wl = g * _L
            best = cost_min(b1, b2, wl, _NA, _N_IN, best_v[pl.ds(wl, _L)])
            best_i = (best * 1024.0).astype(jnp.int32)
            n_bm = best_i & 31
            mcb = best_i >> 10
            if n_cand > _N_IN:
                acc = jnp.zeros((_L,), jnp.float32)
                rr = (_N_IN - _NA) * k2
                for k in range(k2):
                    acc = acc + jnp.abs(b1[rr + k, pl.ds(wl, _L)]
                                        - b2[rr + k, pl.ds(wl, _L)])
                c25 = acc.astype(jnp.int32)
                hit = c25 < mcb
                mskv = hit.astype(jnp.int32)
                mcv = jnp.minimum(c25, mcb)
                idx_tm = jnp.where(hit, _N_IN, n_bm)
            else:
                mskv = jnp.zeros((_L,), jnp.int32)
                mcv = mcb
                idx_tm = n_bm
            row_i = (n_bm * 13) >> 6
            col_i = n_bm - row_i * 5
            sl = pl.ds(wl, _L)
            vv_v[sl] = (2 - row_i).astype(jnp.float32)
            vv_v[pl.ds(w + wl, _L)] = (2 - col_i).astype(jnp.float32)
            msk_v[sl] = mskv
            mcv_v[sl] = mcv
            in_b = idx_tm >= _NA
            rows = (jnp.maximum(idx_tm - _NA, 0)) * k2
            cols = wl + iota
            for k in range(k2):
                val = plsc.load_gather(b1, [rows + k, cols], mask=in_b)
                plsc.store_scatter(
                    tmp_v, [k * w + cols], val, mask=in_b)

    start_in(0, w1a, w2a, sem_a)

    def row_loop(i, _):
        c = 2 * i
        start_in(c + 1, w1b, w2b, sem_b)
        wait_in(w1a, w2a, sem_a)
        compute_a(w1a, w2a)

        @pl.when(i + 1 < rows_per_tec)
        def _():
            start_in(c + 2, w1a, w2a, sem_a)

        wait_in(w1b, w2b, sem_b)
        compute_b(w1b, w2b)
        bh = wid * rows_per_tec + i
        pltpu.sync_copy(vv_v, vv_o.at[pl.ds(bh * 2 * w, 2 * w)])
        pltpu.sync_copy(msk_v, msk_o.at[pl.ds(bh * w, w)])
        pltpu.sync_copy(mcv_v, mcv_o.at[pl.ds(bh * w, w)])
        pltpu.sync_copy(tmp_v, tmp_o.at[pl.ds(bh * k2 * w, k2 * w)])
        return ()

    lax.fori_loop(0, rows_per_tec, row_loop, ())


@jax.jit
def kernel(w1, w2):
    B, H, W, N, K2 = w1.shape
    BH = B * H
    rows_per_tec = BH // _NW
    mesh = plsc.VectorSubcoreMesh(
        core_axis_name="c", subcore_axis_name="s", num_cores=2, num_subcores=16
    )
    f32 = jnp.float32
    i32 = jnp.int32
    out_type = (
        jax.ShapeDtypeStruct((BH * 2 * W,), f32),
        jax.ShapeDtypeStruct((BH * W,), i32),
        jax.ShapeDtypeStruct((BH * W,), i32),
        jax.ShapeDtypeStruct((BH * K2 * W,), f32),
    )
    scratch = (
        pltpu.VMEM((_L * 2,), f32),
        pltpu.VMEM((_NA * K2, W), f32),
        pltpu.VMEM((_NA * K2, W), f32),
        pltpu.VMEM((_NA * K2, W), f32),
        pltpu.VMEM((_NA * K2, W), f32),
        pltpu.VMEM((W,), f32),
        pltpu.VMEM((2 * W,), f32),
        pltpu.VMEM((W,), i32),
        pltpu.VMEM((W,), i32),
        pltpu.VMEM((K2 * W,), f32),
        pltpu.SemaphoreType.DMA,
        pltpu.SemaphoreType.DMA,
    )
    run = pl.kernel(
        functools.partial(_body, n_cand=N, k2=K2, w=W,
                          rows_per_tec=rows_per_tec),
        out_type=out_type,
        mesh=mesh,
        scratch_types=scratch,
        compiler_params=pltpu.CompilerParams(needs_layout_passes=False),
    )
    pcode = np.full((_L * 2,), (1 << 22), np.float32)
    for n in range(_N_IN):
        pcode[n] = ((_PRIO[n] << 5) | n) / 1024.0
    w1t = jnp.transpose(w1, (0, 1, 3, 4, 2)).reshape(BH, N * K2, W)
    w2t = jnp.transpose(w2, (0, 1, 3, 4, 2)).reshape(BH, N * K2, W)
    vv, msk, mcv, tmp = run(jnp.asarray(pcode), w1t, w2t)
    vector = vv.reshape(B, H, 2, W).transpose(0, 1, 3, 2).astype(jnp.float16)
    min_templates = tmp.reshape(B, H, 1, K2, W).transpose(0, 1, 4, 2, 3)
    input_mv_mask = (msk > 0).reshape(B, H, W, 1)
    min_cost_volume = mcv.reshape(B, H, W, 1)
    return (vector, min_templates, input_mv_mask, min_cost_volume)


# speedup vs baseline: 11.4664x; 1.0880x over previous
import functools

import jax
import jax.numpy as jnp
import numpy as np
from jax import lax
from jax.experimental import pallas as pl
from jax.experimental.pallas import tpu as pltpu
from jax.experimental.pallas import tpu_sc as plsc

_SR = 2
_S = 2 * _SR + 1
_N_IN = _S * _S


def _spiral_prio():
    coords = [(0, 0)]
    j = i = 0
    step = 1
    dirs = [(0, 1), (1, 0), (0, -1), (-1, 0)]
    d = 0
    while len(coords) < _S * _S:
        for _ in range(2):
            dj, di = dirs[d]
            for _ in range(step):
                j += dj
                i += di
                if abs(j) <= _SR and abs(i) <= _SR and len(coords) < _S * _S:
                    coords.append((j, i))
            d = (d + 1) % 4
        step += 1
    order = [(jj + _SR) * _S + (ii + _SR) for jj, ii in coords]
    prio = [0] * _N_IN
    for r, n in enumerate(order):
        prio[n] = r
    return prio


_PRIO = _spiral_prio()

_L = 16
_NW = 32


_NA = 13


def _body(pcode_hbm, w1_hbm, w2_hbm, vv_o, msk_o, mcv_o, tmp_o,
          pcode_v, w1a, w2a, w1b, w2b, best_v, vv_v, msk_v, mcv_v, tmp_v,
          sem_a, sem_b,
          *, n_cand, k2, w, rows_per_tec):
    nr = _NA * k2
    wid = lax.axis_index("s") * 2 + lax.axis_index("c")
    iota = lax.broadcasted_iota(jnp.int32, (_L,), 0)
    pltpu.sync_copy(pcode_hbm, pcode_v)

    def start_in(c, b1, b2, sem):
        bh = wid * rows_per_tec + (c >> 1)
        r0 = (c & 1) * nr
        pltpu.make_async_copy(
            w1_hbm.at[bh, pl.ds(r0, nr), :], b1, sem).start()
        pltpu.make_async_copy(
            w2_hbm.at[bh, pl.ds(r0, nr), :], b2, sem).start()

    def wait_in(b1, b2, sem):
        pltpu.make_async_copy(
            w1_hbm.at[0, pl.ds(0, nr), :], b1, sem).wait()
        pltpu.make_async_copy(
            w2_hbm.at[0, pl.ds(0, nr), :], b2, sem).wait()

    code_lo = pcode_v[pl.ds(0, _L)]
    code_hi = pcode_v[pl.ds(_L, _L)]

    def code_of(n):
        bn = jnp.broadcast_to(n, (_L,))
        g_lo = code_lo.at[bn & (_L - 1)].get(mode="promise_in_bounds")
        g_hi = code_hi.at[bn & (_L - 1)].get(mode="promise_in_bounds")
        return jnp.where(bn < _L, g_lo, g_hi)

    def cost_min(b1, b2, wl, n_lo, n_hi, init):
        def nbody(n, best):
            r = (n - n_lo) * k2
            sl = pl.ds(wl, _L)
            pa = [jnp.abs(b1[r + k, sl] - b2[r + k, sl]) for k in range(4)]
            for k in range(4, k2):
                pa[k & 3] = pa[k & 3] + jnp.abs(b1[r + k, sl] - b2[r + k, sl])
            acc = (pa[0] + pa[1]) + (pa[2] + pa[3]) + code_of(n)
            return jnp.minimum(best, acc)

        return lax.fori_loop(n_lo, n_hi, nbody, init)

    def compute_a(b1, b2):
        for g in range(w // _L):
            wl = g * _L
            best = cost_min(b1, b2, wl, 0, _NA,
                            jnp.full((_L,), 3.0e7, jnp.float32))
            best_v[pl.ds(wl, _L)] = best
            na = (best * 1024.0).astype(jnp.int32) & 31
            rows = na * k2
            cols = wl + iota
            for k in range(k2):
                val = plsc.load_gather(b1, [rows + k, cols])
                tmp_v[pl.ds(k * w + wl, _L)] = val

    def compute_b(b1, b2):
        for g in range(w // _L):
            wl = g * _L
            best = cost_min(b1, b2, wl, _NA, _N_IN, best_v[pl.ds(wl, _L)])
            best_i = (best * 1024.0).astype(jnp.int32)
            n_bm = best_i & 31
            mcb = best_i >> 10
            if n_cand > _N_IN:
                acc = jnp.zeros((_L,), jnp.float32)
                rr = (_N_IN - _NA) * k2
                for k in range(k2):
                    acc = acc + jnp.abs(b1[rr + k, pl.ds(wl, _L)]
                                        - b2[rr + k, pl.ds(wl, _L)])
                c25 = acc.astype(jnp.int32)
                hit = c25 < mcb
                mskv = hit.astype(jnp.int32)
                mcv = jnp.minimum(c25, mcb)
                idx_tm = jnp.where(hit, _N_IN, n_bm)
            else:
                mskv = jnp.zeros((_L,), jnp.int32)
                mcv = mcb
                idx_tm = n_bm
            row_i = (n_bm * 13) >> 6
            col_i = n_bm - row_i * 5
            sl = pl.ds(wl, _L)
            vv_v[sl] = (2 - row_i).astype(jnp.float32)
            vv_v[pl.ds(w + wl, _L)] = (2 - col_i).astype(jnp.float32)
            msk_v[sl] = mskv
            mcv_v[sl] = mcv
            in_b = idx_tm >= _NA
            rows = (jnp.maximum(idx_tm - _NA, 0)) * k2
            cols = wl + iota
            for k in range(k2):
                val = plsc.load_gather(b1, [rows + k, cols], mask=in_b)
                plsc.store_scatter(
                    tmp_v, [k * w + cols], val, mask=in_b)

    start_in(0, w1a, w2a, sem_a)

    def row_loop(i, _):
        c = 2 * i
        start_in(c + 1, w1b, w2b, sem_b)
        wait_in(w1a, w2a, sem_a)
        compute_a(w1a, w2a)

        @pl.when(i + 1 < rows_per_tec)
        def _():
            start_in(c + 2, w1a, w2a, sem_a)

        wait_in(w1b, w2b, sem_b)
        compute_b(w1b, w2b)
        bh = wid * rows_per_tec + i
        pltpu.sync_copy(vv_v, vv_o.at[pl.ds(bh * 2 * w, 2 * w)])
        pltpu.sync_copy(msk_v, msk_o.at[pl.ds(bh * w, w)])
        pltpu.sync_copy(mcv_v, mcv_o.at[pl.ds(bh * w, w)])
        pltpu.sync_copy(tmp_v, tmp_o.at[pl.ds(bh * k2 * w, k2 * w)])
        return ()

    lax.fori_loop(0, rows_per_tec, row_loop, ())


@jax.jit
def kernel(w1, w2):
    B, H, W, N, K2 = w1.shape
    BH = B * H
    rows_per_tec = BH // _NW
    mesh = plsc.VectorSubcoreMesh(
        core_axis_name="c", subcore_axis_name="s", num_cores=2, num_subcores=16
    )
    f32 = jnp.float32
    i32 = jnp.int32
    out_type = (
        jax.ShapeDtypeStruct((BH * 2 * W,), f32),
        jax.ShapeDtypeStruct((BH * W,), i32),
        jax.ShapeDtypeStruct((BH * W,), i32),
        jax.ShapeDtypeStruct((BH * K2 * W,), f32),
    )
    scratch = (
        pltpu.VMEM((_L * 2,), f32),
        pltpu.VMEM((_NA * K2, W), f32),
        pltpu.VMEM((_NA * K2, W), f32),
        pltpu.VMEM((_NA * K2, W), f32),
        pltpu.VMEM((_NA * K2, W), f32),
        pltpu.VMEM((W,), f32),
        pltpu.VMEM((2 * W,), f32),
        pltpu.VMEM((W,), i32),
        pltpu.VMEM((W,), i32),
        pltpu.VMEM((K2 * W,), f32),
        pltpu.SemaphoreType.DMA,
        pltpu.SemaphoreType.DMA,
    )
    run = pl.kernel(
        functools.partial(_body, n_cand=N, k2=K2, w=W,
                          rows_per_tec=rows_per_tec),
        out_type=out_type,
        mesh=mesh,
        scratch_types=scratch,
        compiler_params=pltpu.CompilerParams(needs_layout_passes=False),
    )
    pcode = np.full((_L * 2,), (1 << 22), np.float32)
    for n in range(_N_IN):
        pcode[n] = ((_PRIO[n] << 5) | n) / 1024.0
    w1t = jnp.transpose(w1, (0, 1, 3, 4, 2)).reshape(BH, N * K2, W)
    w2t = jnp.transpose(w2, (0, 1, 3, 4, 2)).reshape(BH, N * K2, W)
    vv, msk, mcv, tmp = run(jnp.asarray(pcode), w1t, w2t)
    vector = vv.reshape(B, H, 2, W).transpose(0, 1, 3, 2).astype(jnp.float16)
    min_templates = tmp.reshape(B, H, 1, K2, W).transpose(0, 1, 4, 2, 3)
    input_mv_mask = (msk > 0).reshape(B, H, W, 1)
    min_cost_volume = mcv.reshape(B, H, W, 1)
    return (vector, min_templates, input_mv_mask, min_cost_volume)
